# Initial kernel scaffold; baseline (speedup 1.0000x reference)
import jax
import jax.numpy as jnp
from jax.experimental import pallas as pl


def kernel(f, p, W0, b0, gamma0, beta0, W1, b1, gamma1, beta1, W2, b2, gamma2, beta2):
    raise NotImplementedError("write your pallas kernel here")


# speedup vs baseline: 12.8827x; 12.8827x over previous
import functools

import jax
import jax.numpy as jnp
from jax.experimental import pallas as pl
from jax.experimental.pallas import tpu as pltpu
from jax.experimental.pallas import tpu_sc as plsc

_B = 8
_N = 4096
_S = 1024
_K = 32
_R2 = 0.2 * 0.2
_EPS = 1e-5
_DPAD = 128
_TS = 256
_RT = 1024


def _fps_body(px_ref, py_ref, pz_ref, idx_ref, qx_ref, qy_ref, qz_ref):
    px = px_ref[...]
    py = py_ref[...]
    pz = pz_ref[...]
    iota = jax.lax.broadcasted_iota(jnp.int32, (_B, _N), 1)

    def body(i, carry):
        dists, far = carry
        idx_ref[pl.ds(i, 1), :] = far.reshape(1, _B)
        mask = iota == far
        cx = jnp.sum(jnp.where(mask, px, 0.0), axis=1, keepdims=True)
        cy = jnp.sum(jnp.where(mask, py, 0.0), axis=1, keepdims=True)
        cz = jnp.sum(jnp.where(mask, pz, 0.0), axis=1, keepdims=True)
        qx_ref[pl.ds(i, 1), :] = cx.reshape(1, _B)
        qy_ref[pl.ds(i, 1), :] = cy.reshape(1, _B)
        qz_ref[pl.ds(i, 1), :] = cz.reshape(1, _B)
        dx = px - cx
        dy = py - cy
        dz = pz - cz
        d = (dx * dx + dy * dy) + dz * dz
        dists = jnp.minimum(dists, d)
        m = jnp.max(dists, axis=1, keepdims=True)
        cand = jnp.where(dists == m, iota, _N)
        far = jnp.min(cand, axis=1, keepdims=True)
        return dists, far

    dists0 = jnp.full((_B, _N), 1e10, dtype=jnp.float32)
    far0 = jnp.zeros((_B, 1), dtype=jnp.int32)
    jax.lax.fori_loop(0, _S, body, (dists0, far0))


def _fps(px, py, pz):
    f32 = jnp.float32
    return pl.pallas_call(
        _fps_body,
        out_shape=(
            jax.ShapeDtypeStruct((_S, _B), jnp.int32),
            jax.ShapeDtypeStruct((_S, _B), f32),
            jax.ShapeDtypeStruct((_S, _B), f32),
            jax.ShapeDtypeStruct((_S, _B), f32),
        ),
    )(px, py, pz)


def _cumsum_lanes(x, n):
    sh = 1
    while sh < n:
        shifted = jnp.concatenate(
            [jnp.zeros_like(x[:, :sh]), x[:, : n - sh]], axis=1
        )
        x = x + shifted
        sh *= 2
    return x


def _bq_body(q_ref, px_ref, py_ref, pz_ref, out_ref):
    b = pl.program_id(0)
    q = q_ref[0]
    qx = q[:, 0:1]
    qy = q[:, 1:2]
    qz = q[:, 2:3]
    px = px_ref[0]
    py = py_ref[0]
    pz = pz_ref[0]
    q2 = (qx * qx + qy * qy) + qz * qz
    p2 = (px * px + py * py) + pz * pz
    rp = lambda t: t.astype(jnp.bfloat16).astype(jnp.float32)
    dot = (rp(qx) * rp(px) + rp(qy) * rp(py)) + rp(qz) * rp(pz)
    sqr = (q2 + p2) - 2.0 * dot
    flag = (sqr <= _R2).astype(jnp.float32)
    c = _cumsum_lanes(flag, _N)
    cols = []
    for k in range(_K):
        cnt = jnp.sum(
            jnp.where(c <= jnp.float32(k), jnp.float32(1.0), jnp.float32(0.0)),
            axis=1,
            keepdims=True,
        )
        cols.append(cnt.astype(jnp.int32))
    first = cols[0]
    off = b * _N
    for k in range(_K):
        v = cols[k]
        v = jnp.where(v == _N, first, v) + off
        out_ref[0, :, k : k + 1] = v


def _ball_query(qry, px, py, pz):
    return pl.pallas_call(
        _bq_body,
        grid=(_B, _S // _TS),
        in_specs=[
            pl.BlockSpec((1, _TS, 3), lambda b, s: (b, s, 0)),
            pl.BlockSpec((1, 1, _N), lambda b, s: (b, 0, 0)),
            pl.BlockSpec((1, 1, _N), lambda b, s: (b, 0, 0)),
            pl.BlockSpec((1, 1, _N), lambda b, s: (b, 0, 0)),
        ],
        out_specs=pl.BlockSpec((1, _TS, _K), lambda b, s: (b, s, 0)),
        out_shape=jax.ShapeDtypeStruct((_B, _S, _K), jnp.int32),
    )(qry, px.reshape(_B, 1, _N), py.reshape(_B, 1, _N), pz.reshape(_B, 1, _N))


def _sc_gather(table, idx_flat):
    n_idx = idx_flat.shape[0]
    dim = table.shape[1]
    idx2 = idx_flat.reshape(1, n_idx)
    mesh = plsc.VectorSubcoreMesh(core_axis_name="core", subcore_axis_name="subcore")

    @functools.partial(
        pl.kernel,
        out_type=jax.ShapeDtypeStruct((n_idx, dim), table.dtype),
        mesh=mesh,
    )
    def k(x_hbm, i_hbm, o_hbm):
        def body(i_vmem, o_vmem):
            pltpu.sync_copy(x_hbm.at[i_vmem.at[0]], o_vmem)

        pltpu.emit_pipeline(
            body,
            grid=(n_idx // 128,),
            in_specs=[pl.BlockSpec((1, 128), index_map=lambda i: (0, i))],
            out_specs=[pl.BlockSpec((128, dim), index_map=lambda i: (i, 0))],
            core_axis_name=("core", "subcore"),
            dimension_semantics=(pltpu.PARALLEL,),
        )(i_hbm, o_hbm)

    return k(table, idx2)


def _mlp1_body(x_ref, w_ref, b_ref, q_ref, y_ref, s_ref, ss_ref):
    x = x_ref[...]
    w = w_ref[...]
    y = jnp.dot(x, w, preferred_element_type=jnp.float32)
    q = q_ref[...]
    t = b_ref[...] - (
        (q[:, 0:1] * w_ref[64:65, :] + q[:, 1:2] * w_ref[65:66, :])
        + q[:, 2:3] * w_ref[66:67, :]
    )
    y = y.reshape(_RT // _K, _K, y.shape[-1]) + t[:, None, :]
    y = y.reshape(_RT, y.shape[-1])
    y_ref[...] = y

    @pl.when(pl.program_id(0) == 0)
    def _():
        s_ref[...] = jnp.zeros_like(s_ref)
        ss_ref[...] = jnp.zeros_like(ss_ref)

    s_ref[...] += jnp.sum(y, axis=0, keepdims=True)
    ss_ref[...] += jnp.sum(y * y, axis=0, keepdims=True)


def _mlp1(xg, w0p, b0, qrows):
    n_rows = xg.shape[0]
    cout = w0p.shape[1]
    return pl.pallas_call(
        _mlp1_body,
        grid=(n_rows // _RT,),
        in_specs=[
            pl.BlockSpec((_RT, _DPAD), lambda i: (i, 0)),
            pl.BlockSpec((_DPAD, cout), lambda i: (0, 0)),
            pl.BlockSpec((1, cout), lambda i: (0, 0)),
            pl.BlockSpec((_RT // _K, 3), lambda i: (i, 0)),
        ],
        out_specs=(
            pl.BlockSpec((_RT, cout), lambda i: (i, 0)),
            pl.BlockSpec((1, cout), lambda i: (0, 0)),
            pl.BlockSpec((1, cout), lambda i: (0, 0)),
        ),
        out_shape=(
            jax.ShapeDtypeStruct((n_rows, cout), jnp.float32),
            jax.ShapeDtypeStruct((1, cout), jnp.float32),
            jax.ShapeDtypeStruct((1, cout), jnp.float32),
        ),
    )(xg, w0p, b0, qrows)


def _mlp_mid_body(y_ref, sc_ref, sh_ref, w_ref, o_ref, s_ref, ss_ref):
    y = y_ref[...]
    z = jnp.maximum(y * sc_ref[...] + sh_ref[...], 0.0)
    o = jnp.dot(z, w_ref[...], preferred_element_type=jnp.float32)
    o_ref[...] = o

    @pl.when(pl.program_id(0) == 0)
    def _():
        s_ref[...] = jnp.zeros_like(s_ref)
        ss_ref[...] = jnp.zeros_like(ss_ref)

    s_ref[...] += jnp.sum(o, axis=0, keepdims=True)
    ss_ref[...] += jnp.sum(o * o, axis=0, keepdims=True)


def _mlp_mid(y, scale, shift, w):
    n_rows, cin = y.shape
    cout = w.shape[1]
    return pl.pallas_call(
        _mlp_mid_body,
        grid=(n_rows // _RT,),
        in_specs=[
            pl.BlockSpec((_RT, cin), lambda i: (i, 0)),
            pl.BlockSpec((1, cin), lambda i: (0, 0)),
            pl.BlockSpec((1, cin), lambda i: (0, 0)),
            pl.BlockSpec((cin, cout), lambda i: (0, 0)),
        ],
        out_specs=(
            pl.BlockSpec((_RT, cout), lambda i: (i, 0)),
            pl.BlockSpec((1, cout), lambda i: (0, 0)),
            pl.BlockSpec((1, cout), lambda i: (0, 0)),
        ),
        out_shape=(
            jax.ShapeDtypeStruct((n_rows, cout), jnp.float32),
            jax.ShapeDtypeStruct((1, cout), jnp.float32),
            jax.ShapeDtypeStruct((1, cout), jnp.float32),
        ),
    )(y, scale, shift, w)


def _mlp_last_body(y_ref, sc_ref, sh_ref, o_ref):
    z = jnp.maximum(y_ref[...] * sc_ref[...] + sh_ref[...], 0.0)
    z = z.reshape(_RT // _K, _K, z.shape[-1])
    o_ref[...] = jnp.max(z, axis=1)


def _mlp_last(y, scale, shift):
    n_rows, cin = y.shape
    return pl.pallas_call(
        _mlp_last_body,
        grid=(n_rows // _RT,),
        in_specs=[
            pl.BlockSpec((_RT, cin), lambda i: (i, 0)),
            pl.BlockSpec((1, cin), lambda i: (0, 0)),
            pl.BlockSpec((1, cin), lambda i: (0, 0)),
        ],
        out_specs=pl.BlockSpec((_RT // _K, cin), lambda i: (i, 0)),
        out_shape=jax.ShapeDtypeStruct((n_rows // _K, cin), jnp.float32),
    )(y, scale, shift)


def _bn_coeffs(s, ss, gamma, beta, n):
    mu = s / n
    var = ss / n - mu * mu
    scale = gamma.reshape(1, -1) / jnp.sqrt(var + _EPS)
    shift = beta.reshape(1, -1) - mu * scale
    return scale, shift


def kernel(f, p, W0, b0, gamma0, beta0, W1, b1, gamma1, beta1, W2, b2, gamma2, beta2):
    px = p[:, :, 0]
    py = p[:, :, 1]
    pz = p[:, :, 2]
    _, qxT, qyT, qzT = _fps(px, py, pz)
    qry = jnp.stack([qxT.T, qyT.T, qzT.T], axis=-1)

    grp_idx = _ball_query(qry, px, py, pz)

    table = jnp.concatenate(
        [f, p, jnp.zeros((_B, _N, _DPAD - 67), jnp.float32)], axis=-1
    ).reshape(_B * _N, _DPAD)
    xg = _sc_gather(table, grp_idx.reshape(-1))

    w0p = jnp.concatenate([W0, jnp.zeros((_DPAD - 67, W0.shape[1]), W0.dtype)], axis=0)
    n_rows = _B * _S * _K
    qrows = qry.reshape(_B * _S, 3)

    y1, s1, ss1 = _mlp1(xg, w0p, b0.reshape(1, -1), qrows)
    sc1, sh1 = _bn_coeffs(s1, ss1, gamma0, beta0, n_rows)
    y2, s2, ss2 = _mlp_mid(y1, sc1, sh1, W1)
    sc2, sh2 = _bn_coeffs(s2, ss2, gamma1, beta1, n_rows)
    y3, s3, ss3 = _mlp_mid(y2, sc2, sh2, W2)
    sc3, sh3 = _bn_coeffs(s3, ss3, gamma2, beta2, n_rows)
    x = _mlp_last(y3, sc3, sh3)
    return x.reshape(_B, _S, -1), qry


# speedup vs baseline: 12.9569x; 1.0058x over previous
import functools

import jax
import jax.numpy as jnp
from jax.experimental import pallas as pl
from jax.experimental.pallas import tpu as pltpu
from jax.experimental.pallas import tpu_sc as plsc

_B = 8
_N = 4096
_S = 1024
_K = 32
_R2 = 0.2 * 0.2
_EPS = 1e-5
_DPAD = 128
_TS = 256
_RT = 1024


def _fps_body(px_ref, py_ref, pz_ref, idx_ref, qx_ref, qy_ref, qz_ref):
    px = px_ref[...]
    py = py_ref[...]
    pz = pz_ref[...]
    iota = jax.lax.broadcasted_iota(jnp.int32, (_B, _N), 1)
    pcat = jnp.concatenate([px, py, pz], axis=0)
    iota24 = jax.lax.broadcasted_iota(jnp.int32, (3 * _B, _N), 1)

    def body(i, carry):
        dists, far = carry
        idx_ref[pl.ds(i, 1), :] = far.reshape(1, _B)
        far24 = jnp.concatenate([far, far, far], axis=0)
        csum = jnp.sum(
            jnp.where(iota24 == far24, pcat, 0.0), axis=1, keepdims=True
        )
        cx = csum[0:_B]
        cy = csum[_B : 2 * _B]
        cz = csum[2 * _B : 3 * _B]
        qx_ref[pl.ds(i, 1), :] = cx.reshape(1, _B)
        qy_ref[pl.ds(i, 1), :] = cy.reshape(1, _B)
        qz_ref[pl.ds(i, 1), :] = cz.reshape(1, _B)
        dx = px - cx
        dy = py - cy
        dz = pz - cz
        d = (dx * dx + dy * dy) + dz * dz
        dists = jnp.minimum(dists, d)
        m = jnp.max(dists, axis=1, keepdims=True)
        cand = jnp.where(dists == m, iota, _N)
        far = jnp.min(cand, axis=1, keepdims=True)
        return dists, far

    dists0 = jnp.full((_B, _N), 1e10, dtype=jnp.float32)
    far0 = jnp.zeros((_B, 1), dtype=jnp.int32)
    jax.lax.fori_loop(0, _S, body, (dists0, far0))


def _fps(px, py, pz):
    f32 = jnp.float32
    return pl.pallas_call(
        _fps_body,
        out_shape=(
            jax.ShapeDtypeStruct((_S, _B), jnp.int32),
            jax.ShapeDtypeStruct((_S, _B), f32),
            jax.ShapeDtypeStruct((_S, _B), f32),
            jax.ShapeDtypeStruct((_S, _B), f32),
        ),
    )(px, py, pz)


def _cumsum_lanes(x, n):
    sh = 1
    while sh < n:
        shifted = jnp.concatenate(
            [jnp.zeros_like(x[:, :sh]), x[:, : n - sh]], axis=1
        )
        x = x + shifted
        sh *= 2
    return x


def _bq_body(q_ref, px_ref, py_ref, pz_ref, out_ref):
    b = pl.program_id(0)
    q = q_ref[0]
    qx = q[:, 0:1]
    qy = q[:, 1:2]
    qz = q[:, 2:3]
    px = px_ref[0]
    py = py_ref[0]
    pz = pz_ref[0]
    q2 = (qx * qx + qy * qy) + qz * qz
    p2 = (px * px + py * py) + pz * pz
    rp = lambda t: t.astype(jnp.bfloat16).astype(jnp.float32)
    dot = (rp(qx) * rp(px) + rp(qy) * rp(py)) + rp(qz) * rp(pz)
    sqr = (q2 + p2) - 2.0 * dot
    flag = (sqr <= _R2).astype(jnp.float32)
    c = _cumsum_lanes(flag, _N)
    cols = []
    for k in range(_K):
        cnt = jnp.sum(
            jnp.where(c <= jnp.float32(k), jnp.float32(1.0), jnp.float32(0.0)),
            axis=1,
            keepdims=True,
        )
        cols.append(cnt.astype(jnp.int32))
    first = cols[0]
    off = b * _N
    for k in range(_K):
        v = cols[k]
        v = jnp.where(v == _N, first, v) + off
        out_ref[0, :, k : k + 1] = v


def _ball_query(qry, px, py, pz):
    return pl.pallas_call(
        _bq_body,
        grid=(_B, _S // _TS),
        in_specs=[
            pl.BlockSpec((1, _TS, 3), lambda b, s: (b, s, 0)),
            pl.BlockSpec((1, 1, _N), lambda b, s: (b, 0, 0)),
            pl.BlockSpec((1, 1, _N), lambda b, s: (b, 0, 0)),
            pl.BlockSpec((1, 1, _N), lambda b, s: (b, 0, 0)),
        ],
        out_specs=pl.BlockSpec((1, _TS, _K), lambda b, s: (b, s, 0)),
        out_shape=jax.ShapeDtypeStruct((_B, _S, _K), jnp.int32),
    )(qry, px.reshape(_B, 1, _N), py.reshape(_B, 1, _N), pz.reshape(_B, 1, _N))


def _sc_gather(table, idx_flat):
    n_idx = idx_flat.shape[0]
    dim = table.shape[1]
    idx2 = idx_flat.reshape(1, n_idx)
    mesh = plsc.VectorSubcoreMesh(core_axis_name="core", subcore_axis_name="subcore")

    @functools.partial(
        pl.kernel,
        out_type=jax.ShapeDtypeStruct((n_idx, dim), table.dtype),
        mesh=mesh,
    )
    def k(x_hbm, i_hbm, o_hbm):
        def body(i_vmem, o_vmem):
            pltpu.sync_copy(x_hbm.at[i_vmem.at[0]], o_vmem)

        pltpu.emit_pipeline(
            body,
            grid=(n_idx // 128,),
            in_specs=[pl.BlockSpec((1, 128), index_map=lambda i: (0, i))],
            out_specs=[pl.BlockSpec((128, dim), index_map=lambda i: (i, 0))],
            core_axis_name=("core", "subcore"),
            dimension_semantics=(pltpu.PARALLEL,),
        )(i_hbm, o_hbm)

    return k(table, idx2)


def _mlp1_body(x_ref, w_ref, b_ref, q_ref, y_ref, s_ref, ss_ref):
    x = x_ref[...].astype(jnp.bfloat16)
    w = w_ref[...]
    y = jnp.dot(x, w, preferred_element_type=jnp.float32)
    q = q_ref[...]
    t = b_ref[...] - (
        (
            q[:, 0:1] * w_ref[64:65, :].astype(jnp.float32)
            + q[:, 1:2] * w_ref[65:66, :].astype(jnp.float32)
        )
        + q[:, 2:3] * w_ref[66:67, :].astype(jnp.float32)
    )
    y = y.reshape(_RT // _K, _K, y.shape[-1]) + t[:, None, :]
    y = y.reshape(_RT, y.shape[-1])
    y_ref[...] = y.astype(jnp.bfloat16)

    @pl.when(pl.program_id(0) == 0)
    def _():
        s_ref[...] = jnp.zeros_like(s_ref)
        ss_ref[...] = jnp.zeros_like(ss_ref)

    s_ref[...] += jnp.sum(y, axis=0, keepdims=True)
    ss_ref[...] += jnp.sum(y * y, axis=0, keepdims=True)


def _mlp1(xg, w0p, b0, qrows):
    n_rows = xg.shape[0]
    cout = w0p.shape[1]
    return pl.pallas_call(
        _mlp1_body,
        grid=(n_rows // _RT,),
        in_specs=[
            pl.BlockSpec((_RT, _DPAD), lambda i: (i, 0)),
            pl.BlockSpec((_DPAD, cout), lambda i: (0, 0)),
            pl.BlockSpec((1, cout), lambda i: (0, 0)),
            pl.BlockSpec((_RT // _K, 3), lambda i: (i, 0)),
        ],
        out_specs=(
            pl.BlockSpec((_RT, cout), lambda i: (i, 0)),
            pl.BlockSpec((1, cout), lambda i: (0, 0)),
            pl.BlockSpec((1, cout), lambda i: (0, 0)),
        ),
        out_shape=(
            jax.ShapeDtypeStruct((n_rows, cout), jnp.bfloat16),
            jax.ShapeDtypeStruct((1, cout), jnp.float32),
            jax.ShapeDtypeStruct((1, cout), jnp.float32),
        ),
    )(xg, w0p, b0, qrows)


def _mlp_mid_body(y_ref, sc_ref, sh_ref, w_ref, o_ref, s_ref, ss_ref):
    y = y_ref[...].astype(jnp.float32)
    z = jnp.maximum(y * sc_ref[...] + sh_ref[...], 0.0)
    o = jnp.dot(
        z.astype(jnp.bfloat16), w_ref[...], preferred_element_type=jnp.float32
    )
    o_ref[...] = o.astype(jnp.bfloat16)

    @pl.when(pl.program_id(0) == 0)
    def _():
        s_ref[...] = jnp.zeros_like(s_ref)
        ss_ref[...] = jnp.zeros_like(ss_ref)

    s_ref[...] += jnp.sum(o, axis=0, keepdims=True)
    ss_ref[...] += jnp.sum(o * o, axis=0, keepdims=True)


def _mlp_mid(y, scale, shift, w):
    n_rows, cin = y.shape
    cout = w.shape[1]
    return pl.pallas_call(
        _mlp_mid_body,
        grid=(n_rows // _RT,),
        in_specs=[
            pl.BlockSpec((_RT, cin), lambda i: (i, 0)),
            pl.BlockSpec((1, cin), lambda i: (0, 0)),
            pl.BlockSpec((1, cin), lambda i: (0, 0)),
            pl.BlockSpec((cin, cout), lambda i: (0, 0)),
        ],
        out_specs=(
            pl.BlockSpec((_RT, cout), lambda i: (i, 0)),
            pl.BlockSpec((1, cout), lambda i: (0, 0)),
            pl.BlockSpec((1, cout), lambda i: (0, 0)),
        ),
        out_shape=(
            jax.ShapeDtypeStruct((n_rows, cout), jnp.bfloat16),
            jax.ShapeDtypeStruct((1, cout), jnp.float32),
            jax.ShapeDtypeStruct((1, cout), jnp.float32),
        ),
    )(y, scale, shift, w)


def _mlp_last_body(y_ref, sc_ref, sh_ref, o_ref):
    z = jnp.maximum(
        y_ref[...].astype(jnp.float32) * sc_ref[...] + sh_ref[...], 0.0
    )
    z = z.reshape(_RT // _K, _K, z.shape[-1])
    o_ref[...] = jnp.max(z, axis=1)


def _mlp_last(y, scale, shift):
    n_rows, cin = y.shape
    return pl.pallas_call(
        _mlp_last_body,
        grid=(n_rows // _RT,),
        in_specs=[
            pl.BlockSpec((_RT, cin), lambda i: (i, 0)),
            pl.BlockSpec((1, cin), lambda i: (0, 0)),
            pl.BlockSpec((1, cin), lambda i: (0, 0)),
        ],
        out_specs=pl.BlockSpec((_RT // _K, cin), lambda i: (i, 0)),
        out_shape=jax.ShapeDtypeStruct((n_rows // _K, cin), jnp.float32),
    )(y, scale, shift)


def _bn_coeffs(s, ss, gamma, beta, n):
    mu = s / n
    var = ss / n - mu * mu
    scale = gamma.reshape(1, -1) / jnp.sqrt(var + _EPS)
    shift = beta.reshape(1, -1) - mu * scale
    return scale, shift


def kernel(f, p, W0, b0, gamma0, beta0, W1, b1, gamma1, beta1, W2, b2, gamma2, beta2):
    px = p[:, :, 0]
    py = p[:, :, 1]
    pz = p[:, :, 2]
    _, qxT, qyT, qzT = _fps(px, py, pz)
    qry = jnp.stack([qxT.T, qyT.T, qzT.T], axis=-1)

    grp_idx = _ball_query(qry, px, py, pz)

    table = jnp.concatenate(
        [f, p, jnp.zeros((_B, _N, _DPAD - 67), jnp.float32)], axis=-1
    ).reshape(_B * _N, _DPAD)
    xg = _sc_gather(table, grp_idx.reshape(-1))

    w0p = jnp.concatenate(
        [W0, jnp.zeros((_DPAD - 67, W0.shape[1]), W0.dtype)], axis=0
    ).astype(jnp.bfloat16)
    n_rows = _B * _S * _K
    qrows = qry.reshape(_B * _S, 3)

    y1, s1, ss1 = _mlp1(xg, w0p, b0.reshape(1, -1), qrows)
    sc1, sh1 = _bn_coeffs(s1, ss1, gamma0, beta0, n_rows)
    y2, s2, ss2 = _mlp_mid(y1, sc1, sh1, W1.astype(jnp.bfloat16))
    sc2, sh2 = _bn_coeffs(s2, ss2, gamma1, beta1, n_rows)
    y3, s3, ss3 = _mlp_mid(y2, sc2, sh2, W2.astype(jnp.bfloat16))
    sc3, sh3 = _bn_coeffs(s3, ss3, gamma2, beta2, n_rows)
    x = _mlp_last(y3, sc3, sh3)
    return x.reshape(_B, _S, -1), qry
